# Initial kernel scaffold; baseline (speedup 1.0000x reference)
import jax
import jax.numpy as jnp
from jax.experimental import pallas as pl


def kernel(atom_numbers, physics_encode, onehot_table, W, b):
    raise NotImplementedError("write your pallas kernel here")


# speedup vs baseline: 2.6895x; 2.6895x over previous
import functools

import jax
import jax.numpy as jnp
from jax import lax
from jax.experimental import pallas as pl
from jax.experimental.pallas import tpu as pltpu
from jax.experimental.pallas import tpu_sc as plsc

N_ATOMS = 500000
MAX_ELEMENT = 110
PHYS_DIM = 16
ONEHOT_DIM = 64
ATTR_DIM = PHYS_DIM + ONEHOT_DIM
FEAT_DIM = 128
PAD_ROWS = 128

NC = 2
NS = 16
NW = NC * NS

CHUNK = 128
NCHUNK = -(-N_ATOMS // CHUNK)
ITERS = -(-NCHUNK // NW)

FEAT_BLK = 5000
FEAT_GRID = N_ATOMS // FEAT_BLK


def _prep_body(phys_ref, oh_ref, w_ref, b_ref, attr_ref, feat_ref):
    attr = jnp.concatenate([phys_ref[...], oh_ref[...]], axis=1)
    attr_ref[...] = attr
    feat = lax.dot_general(attr, w_ref[...], (((1,), (1,)), ((), ())),
                           preferred_element_type=jnp.float32)
    feat_ref[...] = feat + b_ref[...]


def _feat_body(idx_ref, tab_ref, out_ref):
    idx = idx_ref[0, 0, :]
    cols = lax.broadcasted_iota(jnp.int32, (FEAT_BLK, FEAT_DIM), 1)
    onehot = (idx[:, None] == cols).astype(jnp.float32)
    out_ref[...] = jnp.dot(onehot, tab_ref[...],
                           preferred_element_type=jnp.float32)


def _sc_gather_body(idx_hbm, tab_hbm, out_hbm, idx_v, rows_v, sem):
    c = lax.axis_index("c")
    s = lax.axis_index("s")
    wid = s * NC + c

    def body(i, carry):
        cid = wid + i * NW

        @pl.when(cid < NCHUNK)
        def _():
            base = pl.multiple_of(
                jnp.minimum(cid * CHUNK, N_ATOMS - CHUNK), 8)
            pltpu.sync_copy(idx_hbm.at[pl.ds(base, CHUNK)], idx_v)
            pltpu.async_copy(tab_hbm.at[idx_v], rows_v, sem).wait()
            pltpu.sync_copy(rows_v, out_hbm.at[pl.ds(base, CHUNK)])

        return carry

    lax.fori_loop(0, ITERS, body, 0)


_sc_gather = functools.partial(
    pl.kernel,
    out_type=jax.ShapeDtypeStruct((N_ATOMS, ATTR_DIM), jnp.float32),
    mesh=plsc.VectorSubcoreMesh(core_axis_name="c", subcore_axis_name="s"),
    compiler_params=pltpu.CompilerParams(use_tc_tiling_on_sc=False),
    scratch_types=[
        pltpu.VMEM((CHUNK,), jnp.int32),
        pltpu.VMEM((CHUNK, ATTR_DIM), jnp.float32),
        pltpu.SemaphoreType.DMA,
    ],
)(_sc_gather_body)


def kernel(atom_numbers, physics_encode, onehot_table, W, b):
    pad = PAD_ROWS - MAX_ELEMENT
    phys_p = jnp.pad(physics_encode, ((0, pad), (0, 0)))
    oh_p = jnp.pad(onehot_table, ((0, pad), (0, 0)))
    idx = atom_numbers.astype(jnp.int32)

    attr_tab, feat_tab = pl.pallas_call(
        _prep_body,
        out_shape=(
            jax.ShapeDtypeStruct((PAD_ROWS, ATTR_DIM), jnp.float32),
            jax.ShapeDtypeStruct((PAD_ROWS, FEAT_DIM), jnp.float32),
        ),
    )(phys_p, oh_p, W, b.reshape(1, FEAT_DIM))

    features = pl.pallas_call(
        _feat_body,
        grid=(FEAT_GRID,),
        in_specs=[
            pl.BlockSpec((1, 1, FEAT_BLK), lambda i: (i, 0, 0)),
            pl.BlockSpec((PAD_ROWS, FEAT_DIM), lambda i: (0, 0)),
        ],
        out_specs=pl.BlockSpec((FEAT_BLK, FEAT_DIM), lambda i: (i, 0)),
        out_shape=jax.ShapeDtypeStruct((N_ATOMS, FEAT_DIM), jnp.float32),
    )(idx.reshape(FEAT_GRID, 1, FEAT_BLK), feat_tab)

    atom_attrs = _sc_gather(idx, attr_tab)
    return (atom_attrs, features)
